# Initial kernel scaffold; baseline (speedup 1.0000x reference)
import jax
import jax.numpy as jnp
from jax.experimental import pallas as pl


def kernel(pos_undirected, seed, edge_index, edge_weight, graph_ids, mlp_W1, mlp_b1, mlp_g1, mlp_be1, mlp_W2, mlp_b2, app_g, app_be, out_g, out_be, pred_W, pred_b):
    raise NotImplementedError("write your pallas kernel here")


# speedup vs baseline: 2.2901x; 2.2901x over previous
import functools

import jax
import jax.numpy as jnp
from jax import lax
from jax.experimental import pallas as pl
from jax.experimental.pallas import tpu as pltpu
from jax.experimental.pallas import tpu_sc as plsc

N = 10000
E = 160000
POS = 128
HID = 512
OUT = 128
NLAYERS = 6
NGRAPH = 256
BN_EPS = 1e-5

LANES = 128
NTILES = 16
EPT = E // NTILES
BLK = 128
NBLK = -(-EPT // BLK)
EPT_PAD = NBLK * BLK
N_PAD = 10240
ROWS_PER_TILE = N_PAD // NTILES
BN_BLK = 1000
GRID_N = N // BN_BLK


@functools.partial(jax.jit, static_argnums=(0,))
def _sc_aggregate(nch, h_c, src_b, dst_b, wb_b, zero_rows):
    mesh = plsc.VectorSubcoreMesh(core_axis_name="c", subcore_axis_name="s")

    @functools.partial(
        pl.kernel,
        mesh=mesh,
        out_type=jax.ShapeDtypeStruct((nch, N_PAD, LANES), jnp.float32),
        scratch_types=[
            pltpu.VMEM((1, BLK), jnp.int32),
            pltpu.VMEM((1, BLK), jnp.int32),
            pltpu.VMEM((BLK, 16), jnp.float32),
            pltpu.VMEM((BLK, LANES), jnp.float32),
            pltpu.VMEM_SHARED((N_PAD, LANES), jnp.float32),
            pltpu.SemaphoreType.DMA,
        ],
    )
    def agg_kernel(h_hbm, src_hbm, dst_hbm, wb_hbm, zero_hbm, agg_hbm,
                   srcv, dstv, wbv, rowsv, slab, sem):
        cid = lax.axis_index("c")
        sid = lax.axis_index("s")
        base = sid * ROWS_PER_TILE
        for ch in range(nch):
            @pl.when(cid == (ch % 2))
            def _round(ch=ch):
                pltpu.sync_copy(zero_hbm, slab.at[pl.ds(base, ROWS_PER_TILE)])
                plsc.subcore_barrier()

                def body(j, carry):
                    pltpu.sync_copy(src_hbm.at[sid, j], srcv.at[0])
                    pltpu.sync_copy(dst_hbm.at[sid, j], dstv.at[0])
                    pltpu.sync_copy(wb_hbm.at[sid, j], wbv)
                    pltpu.async_copy(h_hbm.at[ch].at[srcv.at[0]], rowsv,
                                     sem).wait()

                    def rbody(r, c2):
                        w16 = wbv[r, :]
                        for g in range(LANES // 16):
                            sl = pl.ds(g * 16, 16)
                            rowsv[r, sl] = rowsv[r, sl] * w16
                        return c2

                    lax.fori_loop(0, BLK, rbody, 0)
                    pltpu.sync_copy(rowsv, slab.at[dstv.at[0]], add=True)
                    return carry

                lax.fori_loop(0, NBLK, body, 0)
                plsc.subcore_barrier()
                pltpu.sync_copy(
                    slab.at[pl.ds(base, ROWS_PER_TILE)],
                    agg_hbm.at[ch].at[pl.ds(base, ROWS_PER_TILE)])
                plsc.subcore_barrier()

    return agg_kernel(h_c, src_b, dst_b, wb_b, zero_rows)


def _p1(h_c, agg_c, w1r, b1):
    nch = h_c.shape[0]

    def body(h_ref, a_ref, w_ref, b_ref, u_ref, st_ref):
        i = pl.program_id(0)
        acc = jnp.dot(h_ref[0] + a_ref[0], w_ref[0],
                      preferred_element_type=jnp.float32)
        for c in range(1, nch):
            acc += jnp.dot(h_ref[c] + a_ref[c], w_ref[c],
                           preferred_element_type=jnp.float32)
        u = acc + b_ref[:]
        u_ref[:] = u
        blk = jnp.concatenate(
            [jnp.sum(u, axis=0, keepdims=True),
             jnp.sum(u * u, axis=0, keepdims=True)], axis=0)

        @pl.when(i == 0)
        def _():
            st_ref[:] = blk

        @pl.when(i > 0)
        def _():
            st_ref[:] = st_ref[:] + blk

    return pl.pallas_call(
        body,
        grid=(GRID_N,),
        in_specs=[
            pl.BlockSpec((nch, BN_BLK, LANES), lambda i: (0, i, 0)),
            pl.BlockSpec((nch, BN_BLK, LANES), lambda i: (0, i, 0)),
            pl.BlockSpec((nch, LANES, HID), lambda i: (0, 0, 0)),
            pl.BlockSpec((1, HID), lambda i: (0, 0)),
        ],
        out_specs=[
            pl.BlockSpec((BN_BLK, HID), lambda i: (i, 0)),
            pl.BlockSpec((2, HID), lambda i: (0, 0)),
        ],
        out_shape=[
            jax.ShapeDtypeStruct((N, HID), jnp.float32),
            jax.ShapeDtypeStruct((2, HID), jnp.float32),
        ],
    )(h_c, agg_c, w1r, b1)


def _norm(x, st_ref, g_ref, be_ref):
    m = st_ref[0:1, :] * (1.0 / N)
    var = st_ref[1:2, :] * (1.0 / N) - m * m
    xn = (x - m) * lax.rsqrt(var + BN_EPS) * g_ref[:] + be_ref[:]
    return jnp.maximum(xn, 0.0)


def _p2(u, st1, g1, be1, w2, b2):
    def body(u_ref, st_ref, g_ref, be_ref, w_ref, b_ref, v_ref, st2_ref):
        i = pl.program_id(0)
        x = _norm(u_ref[:], st_ref, g_ref, be_ref)
        v = jnp.dot(x, w_ref[:], preferred_element_type=jnp.float32) + b_ref[:]
        v_ref[:] = v
        blk = jnp.concatenate(
            [jnp.sum(v, axis=0, keepdims=True),
             jnp.sum(v * v, axis=0, keepdims=True)], axis=0)

        @pl.when(i == 0)
        def _():
            st2_ref[:] = blk

        @pl.when(i > 0)
        def _():
            st2_ref[:] = st2_ref[:] + blk

    return pl.pallas_call(
        body,
        grid=(GRID_N,),
        in_specs=[
            pl.BlockSpec((BN_BLK, HID), lambda i: (i, 0)),
            pl.BlockSpec((2, HID), lambda i: (0, 0)),
            pl.BlockSpec((1, HID), lambda i: (0, 0)),
            pl.BlockSpec((1, HID), lambda i: (0, 0)),
            pl.BlockSpec((HID, HID), lambda i: (0, 0)),
            pl.BlockSpec((1, HID), lambda i: (0, 0)),
        ],
        out_specs=[
            pl.BlockSpec((BN_BLK, HID), lambda i: (i, 0)),
            pl.BlockSpec((2, HID), lambda i: (0, 0)),
        ],
        out_shape=[
            jax.ShapeDtypeStruct((N, HID), jnp.float32),
            jax.ShapeDtypeStruct((2, HID), jnp.float32),
        ],
    )(u, st1, g1, be1, w2, b2)


def _p3(v, st2, ag, abe):
    def body(v_ref, st_ref, g_ref, be_ref, x2_ref, st3_ref):
        i = pl.program_id(0)
        x2 = _norm(v_ref[:], st_ref, g_ref, be_ref)
        x2_ref[:] = x2
        blk = jnp.concatenate(
            [jnp.sum(x2, axis=0, keepdims=True),
             jnp.sum(x2 * x2, axis=0, keepdims=True)], axis=0)

        @pl.when(i == 0)
        def _():
            st3_ref[:] = blk

        @pl.when(i > 0)
        def _():
            st3_ref[:] = st3_ref[:] + blk

    return pl.pallas_call(
        body,
        grid=(GRID_N,),
        in_specs=[
            pl.BlockSpec((BN_BLK, HID), lambda i: (i, 0)),
            pl.BlockSpec((2, HID), lambda i: (0, 0)),
            pl.BlockSpec((1, HID), lambda i: (0, 0)),
            pl.BlockSpec((1, HID), lambda i: (0, 0)),
        ],
        out_specs=[
            pl.BlockSpec((BN_BLK, HID), lambda i: (i, 0)),
            pl.BlockSpec((2, HID), lambda i: (0, 0)),
        ],
        out_shape=[
            jax.ShapeDtypeStruct((N, HID), jnp.float32),
            jax.ShapeDtypeStruct((2, HID), jnp.float32),
        ],
    )(v, st2, ag, abe)


def _p4(x2, st3, og, obe, pw, y_in):
    nch = HID // LANES

    def body(x_ref, st_ref, g_ref, be_ref, pw_ref, y_ref, h4_ref, yo_ref):
        hn = _norm(x_ref[:], st_ref, g_ref, be_ref)
        for c in range(nch):
            h4_ref[c] = hn[:, c * LANES:(c + 1) * LANES]
        yo_ref[:] = y_ref[:] + jnp.dot(hn, pw_ref[:],
                                       preferred_element_type=jnp.float32)

    return pl.pallas_call(
        body,
        grid=(GRID_N,),
        in_specs=[
            pl.BlockSpec((BN_BLK, HID), lambda i: (i, 0)),
            pl.BlockSpec((2, HID), lambda i: (0, 0)),
            pl.BlockSpec((1, HID), lambda i: (0, 0)),
            pl.BlockSpec((1, HID), lambda i: (0, 0)),
            pl.BlockSpec((HID, OUT), lambda i: (0, 0)),
            pl.BlockSpec((BN_BLK, OUT), lambda i: (i, 0)),
        ],
        out_specs=[
            pl.BlockSpec((nch, BN_BLK, LANES), lambda i: (0, i, 0)),
            pl.BlockSpec((BN_BLK, OUT), lambda i: (i, 0)),
        ],
        out_shape=[
            jax.ShapeDtypeStruct((nch, N, LANES), jnp.float32),
            jax.ShapeDtypeStruct((N, OUT), jnp.float32),
        ],
        input_output_aliases={5: 1},
    )(x2, st3, og, obe, pw, y_in)


def _y0(h0c, pw0r):
    nch = h0c.shape[0]

    def body(h_ref, w_ref, y_ref):
        acc = jnp.dot(h_ref[0], w_ref[0], preferred_element_type=jnp.float32)
        for c in range(1, nch):
            acc += jnp.dot(h_ref[c], w_ref[c],
                           preferred_element_type=jnp.float32)
        y_ref[:] = acc

    return pl.pallas_call(
        body,
        grid=(GRID_N,),
        in_specs=[
            pl.BlockSpec((nch, BN_BLK, LANES), lambda i: (0, i, 0)),
            pl.BlockSpec((nch, LANES, OUT), lambda i: (0, 0, 0)),
        ],
        out_specs=pl.BlockSpec((BN_BLK, OUT), lambda i: (i, 0)),
        out_shape=jax.ShapeDtypeStruct((N, OUT), jnp.float32),
    )(h0c, pw0r)


def _readout(y, gid2, sumb):
    def body(y_ref, g_ref, sb_ref, s_ref):
        i = pl.program_id(0)
        seg = lax.broadcasted_iota(jnp.int32, (NGRAPH, BN_BLK), 0)
        p = (g_ref[0] == seg).astype(jnp.float32)
        blk = jnp.dot(p, y_ref[:], preferred_element_type=jnp.float32)

        @pl.when(i == 0)
        def _():
            s_ref[:] = blk + sb_ref[:]

        @pl.when(i > 0)
        def _():
            s_ref[:] = s_ref[:] + blk

    return pl.pallas_call(
        body,
        grid=(GRID_N,),
        in_specs=[
            pl.BlockSpec((BN_BLK, OUT), lambda i: (i, 0)),
            pl.BlockSpec((1, 1, BN_BLK), lambda i: (i, 0, 0)),
            pl.BlockSpec((1, OUT), lambda i: (0, 0)),
        ],
        out_specs=pl.BlockSpec((NGRAPH, OUT), lambda i: (0, 0)),
        out_shape=jax.ShapeDtypeStruct((NGRAPH, OUT), jnp.float32),
    )(y, gid2, sumb)


def kernel(pos_undirected, seed, edge_index, edge_weight, graph_ids,
           mlp_W1, mlp_b1, mlp_g1, mlp_be1, mlp_W2, mlp_b2,
           app_g, app_be, out_g, out_be, pred_W, pred_b):
    f32 = jnp.float32

    h0 = jnp.concatenate([pos_undirected, seed.astype(f32)[:, None]], axis=1)
    h0p = jnp.pad(h0, ((0, 0), (0, 2 * LANES - (POS + 1))))
    h0c = jnp.transpose(h0p.reshape(N, 2, LANES), (1, 0, 2))

    pad = EPT_PAD * NTILES - E
    ar = jnp.arange(pad, dtype=jnp.int32)
    src_b = jnp.concatenate([edge_index[0], ar]).reshape(NTILES, NBLK, BLK)
    dst_b = jnp.concatenate([edge_index[1], ar]).reshape(NTILES, NBLK, BLK)
    w_p = jnp.concatenate([edge_weight, jnp.zeros((pad,), f32)])
    wb_b = jnp.broadcast_to(
        w_p.reshape(NTILES, NBLK, BLK)[..., None], (NTILES, NBLK, BLK, 16))
    wb_b = jnp.asarray(wb_b)
    zero_rows = jnp.zeros((ROWS_PER_TILE, LANES), f32)

    def r1(x):
        return x.reshape(1, -1)

    w1r0 = jnp.zeros((2 * LANES, HID), f32).at[:POS + 1, :].set(mlp_W1[0])
    w1r = [jnp.transpose(w1r0.reshape(2, LANES, HID), (0, 1, 2))]
    for i in range(1, NLAYERS - 1):
        w1r.append(mlp_W1[i].reshape(HID // LANES, LANES, HID))

    pw0 = jnp.zeros((2 * LANES, OUT), f32).at[:POS + 1, :].set(pred_W[0])
    pw0r = pw0.reshape(2, LANES, OUT)

    gid2 = graph_ids.reshape(GRID_N, 1, BN_BLK)
    sumb = r1(sum(pred_b[1:], pred_b[0]))

    h_c = h0c
    y = _y0(h0c, pw0r)
    for i in range(NLAYERS - 1):
        nch = h_c.shape[0]
        agg = _sc_aggregate(nch, h_c, src_b, dst_b, wb_b, zero_rows)
        u, st1 = _p1(h_c, agg, w1r[i], r1(mlp_b1[i]))
        v, st2 = _p2(u, st1, r1(mlp_g1[i]), r1(mlp_be1[i]),
                     mlp_W2[i], r1(mlp_b2[i]))
        x2, st3 = _p3(v, st2, r1(app_g[i]), r1(app_be[i]))
        h_c, y = _p4(x2, st3, r1(out_g[i]), r1(out_be[i]), pred_W[i + 1], y)

    return _readout(y, gid2, sumb)


# speedup vs baseline: 3.4872x; 1.5227x over previous
import functools

import jax
import jax.numpy as jnp
from jax import lax
from jax.experimental import pallas as pl
from jax.experimental.pallas import tpu as pltpu
from jax.experimental.pallas import tpu_sc as plsc

N = 10000
E = 160000
POS = 128
HID = 512
OUT = 128
NLAYERS = 6
NGRAPH = 256
BN_EPS = 1e-5

LANES = 128
NTILES = 16
EPT = E // NTILES
BLK = 64
UNROLL = 6
NBLK = ((EPT + BLK * UNROLL - 1) // (BLK * UNROLL)) * UNROLL
NGRP = NBLK // UNROLL
EPT_PAD = NBLK * BLK
N_PAD = 10240
ROWS_PER_TILE = N_PAD // NTILES
BN_BLK = 1000
GRID_N = N // BN_BLK


@functools.partial(jax.jit, static_argnums=(0,))
def _sc_aggregate(nch, h_c, src_b, dst_b, wb_b, zero_rows):
    mesh = plsc.VectorSubcoreMesh(core_axis_name="c", subcore_axis_name="s")

    @functools.partial(
        pl.kernel,
        mesh=mesh,
        out_type=jax.ShapeDtypeStruct((nch, N_PAD, LANES), jnp.float32),
        scratch_types=[
            pltpu.VMEM((3, BLK), jnp.int32),
            pltpu.VMEM((3, BLK), jnp.int32),
            pltpu.VMEM((2, BLK, 16), jnp.float32),
            pltpu.VMEM((2, BLK, LANES), jnp.float32),
            pltpu.VMEM_SHARED((N_PAD, LANES), jnp.float32),
            pltpu.SemaphoreType.DMA,
            pltpu.SemaphoreType.DMA,
            pltpu.SemaphoreType.DMA,
        ],
    )
    def agg_kernel(h_hbm, src_hbm, dst_hbm, wb_hbm, zero_hbm, agg_hbm,
                   srcv, dstv, wbv, rowsv, slab, sem_i, sem_g, sem_s):
        cid = lax.axis_index("c")
        sid = lax.axis_index("s")
        base = sid * ROWS_PER_TILE
        for ch in range(nch):
            @pl.when(cid == (ch % 2))
            def _round(ch=ch):
                hc = h_hbm.at[ch]
                pltpu.sync_copy(zero_hbm, slab.at[pl.ds(base, ROWS_PER_TILE)])
                plsc.subcore_barrier()

                def idx_dmas(j, i3, i2):
                    return (
                        pltpu.make_async_copy(src_hbm.at[sid, j],
                                              srcv.at[i3], sem_i),
                        pltpu.make_async_copy(dst_hbm.at[sid, j],
                                              dstv.at[i3], sem_i),
                        pltpu.make_async_copy(wb_hbm.at[sid, j],
                                              wbv.at[i2], sem_i),
                    )

                def gather(j, i3, b2):
                    return pltpu.make_async_copy(
                        hc.at[srcv.at[i3]], rowsv.at[b2], sem_g)

                def scatter(i3, b2):
                    return pltpu.make_async_copy(
                        rowsv.at[b2], slab.at[dstv.at[i3]], sem_s)

                def scale(b2):
                    def rbody(i, c2):
                        for k in range(2):
                            r = 2 * i + k
                            w16 = wbv[b2, r, :]
                            for g in range(LANES // 16):
                                sl = pl.ds(g * 16, 16)
                                rowsv[b2, r, sl] = rowsv[b2, r, sl] * w16
                        return c2

                    lax.fori_loop(0, BLK // 2, rbody, 0)

                def stage(j, u):
                    i3, b2 = u % 3, u % 2
                    i3n, b2n = (u + 1) % 3, (u + 1) % 2
                    first = isinstance(j, int) and j == 0
                    has_next = not (isinstance(j, int) and j + 1 >= NBLK)
                    has_next2 = not (isinstance(j, int) and j + 2 >= NBLK)
                    if not first:
                        scatter((u + 2) % 3, b2n).wait()
                    if has_next:
                        for d in idx_dmas(j + 1, i3n, b2n):
                            d.wait()
                        gather(j + 1, i3n, b2n).start()
                    gather(j, i3, b2).wait()
                    scale(b2)
                    scatter(i3, b2).start(add=True)
                    if has_next2:
                        for d in idx_dmas(j + 2, (u + 2) % 3, b2):
                            d.start()

                for d in idx_dmas(0, 0, 0) + idx_dmas(1, 1, 1):
                    d.start()
                for d in idx_dmas(0, 0, 0):
                    d.wait()
                gather(0, 0, 0).start()
                for u in range(UNROLL):
                    stage(u, u)

                def body(g, carry):
                    for u in range(UNROLL):
                        stage(g * UNROLL + u, u)
                    return carry

                lax.fori_loop(1, NGRP - 1, body, 0)
                for u in range(UNROLL):
                    stage((NGRP - 1) * UNROLL + u, u)
                scatter((UNROLL - 1) % 3, (UNROLL - 1) % 2).wait()

                plsc.subcore_barrier()
                pltpu.sync_copy(
                    slab.at[pl.ds(base, ROWS_PER_TILE)],
                    agg_hbm.at[ch].at[pl.ds(base, ROWS_PER_TILE)])
                plsc.subcore_barrier()

    return agg_kernel(h_c, src_b, dst_b, wb_b, zero_rows)


def _p1(h_c, agg_c, w1r, b1):
    nch = h_c.shape[0]

    def body(h_ref, a_ref, w_ref, b_ref, u_ref, st_ref):
        i = pl.program_id(0)
        acc = jnp.dot(h_ref[0] + a_ref[0], w_ref[0],
                      preferred_element_type=jnp.float32)
        for c in range(1, nch):
            acc += jnp.dot(h_ref[c] + a_ref[c], w_ref[c],
                           preferred_element_type=jnp.float32)
        u = acc + b_ref[:]
        u_ref[:] = u
        blk = jnp.concatenate(
            [jnp.sum(u, axis=0, keepdims=True),
             jnp.sum(u * u, axis=0, keepdims=True)], axis=0)

        @pl.when(i == 0)
        def _():
            st_ref[:] = blk

        @pl.when(i > 0)
        def _():
            st_ref[:] = st_ref[:] + blk

    return pl.pallas_call(
        body,
        grid=(GRID_N,),
        in_specs=[
            pl.BlockSpec((nch, BN_BLK, LANES), lambda i: (0, i, 0)),
            pl.BlockSpec((nch, BN_BLK, LANES), lambda i: (0, i, 0)),
            pl.BlockSpec((nch, LANES, HID), lambda i: (0, 0, 0)),
            pl.BlockSpec((1, HID), lambda i: (0, 0)),
        ],
        out_specs=[
            pl.BlockSpec((BN_BLK, HID), lambda i: (i, 0)),
            pl.BlockSpec((2, HID), lambda i: (0, 0)),
        ],
        out_shape=[
            jax.ShapeDtypeStruct((N, HID), jnp.float32),
            jax.ShapeDtypeStruct((2, HID), jnp.float32),
        ],
    )(h_c, agg_c, w1r, b1)


def _norm(x, st_ref, g_ref, be_ref):
    m = st_ref[0:1, :] * (1.0 / N)
    var = st_ref[1:2, :] * (1.0 / N) - m * m
    xn = (x - m) * lax.rsqrt(var + BN_EPS) * g_ref[:] + be_ref[:]
    return jnp.maximum(xn, 0.0)


def _p2(u, st1, g1, be1, w2, b2):
    def body(u_ref, st_ref, g_ref, be_ref, w_ref, b_ref, v_ref, st2_ref):
        i = pl.program_id(0)
        x = _norm(u_ref[:], st_ref, g_ref, be_ref)
        v = jnp.dot(x, w_ref[:], preferred_element_type=jnp.float32) + b_ref[:]
        v_ref[:] = v
        blk = jnp.concatenate(
            [jnp.sum(v, axis=0, keepdims=True),
             jnp.sum(v * v, axis=0, keepdims=True)], axis=0)

        @pl.when(i == 0)
        def _():
            st2_ref[:] = blk

        @pl.when(i > 0)
        def _():
            st2_ref[:] = st2_ref[:] + blk

    return pl.pallas_call(
        body,
        grid=(GRID_N,),
        in_specs=[
            pl.BlockSpec((BN_BLK, HID), lambda i: (i, 0)),
            pl.BlockSpec((2, HID), lambda i: (0, 0)),
            pl.BlockSpec((1, HID), lambda i: (0, 0)),
            pl.BlockSpec((1, HID), lambda i: (0, 0)),
            pl.BlockSpec((HID, HID), lambda i: (0, 0)),
            pl.BlockSpec((1, HID), lambda i: (0, 0)),
        ],
        out_specs=[
            pl.BlockSpec((BN_BLK, HID), lambda i: (i, 0)),
            pl.BlockSpec((2, HID), lambda i: (0, 0)),
        ],
        out_shape=[
            jax.ShapeDtypeStruct((N, HID), jnp.float32),
            jax.ShapeDtypeStruct((2, HID), jnp.float32),
        ],
    )(u, st1, g1, be1, w2, b2)


def _p3(v, st2, ag, abe):
    def body(v_ref, st_ref, g_ref, be_ref, x2_ref, st3_ref):
        i = pl.program_id(0)
        x2 = _norm(v_ref[:], st_ref, g_ref, be_ref)
        x2_ref[:] = x2
        blk = jnp.concatenate(
            [jnp.sum(x2, axis=0, keepdims=True),
             jnp.sum(x2 * x2, axis=0, keepdims=True)], axis=0)

        @pl.when(i == 0)
        def _():
            st3_ref[:] = blk

        @pl.when(i > 0)
        def _():
            st3_ref[:] = st3_ref[:] + blk

    return pl.pallas_call(
        body,
        grid=(GRID_N,),
        in_specs=[
            pl.BlockSpec((BN_BLK, HID), lambda i: (i, 0)),
            pl.BlockSpec((2, HID), lambda i: (0, 0)),
            pl.BlockSpec((1, HID), lambda i: (0, 0)),
            pl.BlockSpec((1, HID), lambda i: (0, 0)),
        ],
        out_specs=[
            pl.BlockSpec((BN_BLK, HID), lambda i: (i, 0)),
            pl.BlockSpec((2, HID), lambda i: (0, 0)),
        ],
        out_shape=[
            jax.ShapeDtypeStruct((N, HID), jnp.float32),
            jax.ShapeDtypeStruct((2, HID), jnp.float32),
        ],
    )(v, st2, ag, abe)


def _p4(x2, st3, og, obe, pw, y_in):
    nch = HID // LANES

    def body(x_ref, st_ref, g_ref, be_ref, pw_ref, y_ref, h4_ref, yo_ref):
        hn = _norm(x_ref[:], st_ref, g_ref, be_ref)
        for c in range(nch):
            h4_ref[c] = hn[:, c * LANES:(c + 1) * LANES]
        yo_ref[:] = y_ref[:] + jnp.dot(hn, pw_ref[:],
                                       preferred_element_type=jnp.float32)

    return pl.pallas_call(
        body,
        grid=(GRID_N,),
        in_specs=[
            pl.BlockSpec((BN_BLK, HID), lambda i: (i, 0)),
            pl.BlockSpec((2, HID), lambda i: (0, 0)),
            pl.BlockSpec((1, HID), lambda i: (0, 0)),
            pl.BlockSpec((1, HID), lambda i: (0, 0)),
            pl.BlockSpec((HID, OUT), lambda i: (0, 0)),
            pl.BlockSpec((BN_BLK, OUT), lambda i: (i, 0)),
        ],
        out_specs=[
            pl.BlockSpec((nch, BN_BLK, LANES), lambda i: (0, i, 0)),
            pl.BlockSpec((BN_BLK, OUT), lambda i: (i, 0)),
        ],
        out_shape=[
            jax.ShapeDtypeStruct((nch, N, LANES), jnp.float32),
            jax.ShapeDtypeStruct((N, OUT), jnp.float32),
        ],
        input_output_aliases={5: 1},
    )(x2, st3, og, obe, pw, y_in)


def _y0(h0c, pw0r):
    nch = h0c.shape[0]

    def body(h_ref, w_ref, y_ref):
        acc = jnp.dot(h_ref[0], w_ref[0], preferred_element_type=jnp.float32)
        for c in range(1, nch):
            acc += jnp.dot(h_ref[c], w_ref[c],
                           preferred_element_type=jnp.float32)
        y_ref[:] = acc

    return pl.pallas_call(
        body,
        grid=(GRID_N,),
        in_specs=[
            pl.BlockSpec((nch, BN_BLK, LANES), lambda i: (0, i, 0)),
            pl.BlockSpec((nch, LANES, OUT), lambda i: (0, 0, 0)),
        ],
        out_specs=pl.BlockSpec((BN_BLK, OUT), lambda i: (i, 0)),
        out_shape=jax.ShapeDtypeStruct((N, OUT), jnp.float32),
    )(h0c, pw0r)


def _readout(y, gid2, sumb):
    def body(y_ref, g_ref, sb_ref, s_ref):
        i = pl.program_id(0)
        seg = lax.broadcasted_iota(jnp.int32, (NGRAPH, BN_BLK), 0)
        p = (g_ref[0] == seg).astype(jnp.float32)
        blk = jnp.dot(p, y_ref[:], preferred_element_type=jnp.float32)

        @pl.when(i == 0)
        def _():
            s_ref[:] = blk + sb_ref[:]

        @pl.when(i > 0)
        def _():
            s_ref[:] = s_ref[:] + blk

    return pl.pallas_call(
        body,
        grid=(GRID_N,),
        in_specs=[
            pl.BlockSpec((BN_BLK, OUT), lambda i: (i, 0)),
            pl.BlockSpec((1, 1, BN_BLK), lambda i: (i, 0, 0)),
            pl.BlockSpec((1, OUT), lambda i: (0, 0)),
        ],
        out_specs=pl.BlockSpec((NGRAPH, OUT), lambda i: (0, 0)),
        out_shape=jax.ShapeDtypeStruct((NGRAPH, OUT), jnp.float32),
    )(y, gid2, sumb)


def kernel(pos_undirected, seed, edge_index, edge_weight, graph_ids,
           mlp_W1, mlp_b1, mlp_g1, mlp_be1, mlp_W2, mlp_b2,
           app_g, app_be, out_g, out_be, pred_W, pred_b):
    f32 = jnp.float32

    h0 = jnp.concatenate([pos_undirected, seed.astype(f32)[:, None]], axis=1)
    h0p = jnp.pad(h0, ((0, 0), (0, 2 * LANES - (POS + 1))))
    h0c = jnp.transpose(h0p.reshape(N, 2, LANES), (1, 0, 2))

    pad = EPT_PAD * NTILES - E
    ar = jnp.arange(pad, dtype=jnp.int32)
    src_b = jnp.concatenate([edge_index[0], ar]).reshape(NTILES, NBLK, BLK)
    dst_b = jnp.concatenate([edge_index[1], ar]).reshape(NTILES, NBLK, BLK)
    w_p = jnp.concatenate([edge_weight, jnp.zeros((pad,), f32)])
    wb_b = jnp.broadcast_to(
        w_p.reshape(NTILES, NBLK, BLK)[..., None], (NTILES, NBLK, BLK, 16))
    wb_b = jnp.asarray(wb_b)
    zero_rows = jnp.zeros((ROWS_PER_TILE, LANES), f32)

    def r1(x):
        return x.reshape(1, -1)

    w1r0 = jnp.zeros((2 * LANES, HID), f32).at[:POS + 1, :].set(mlp_W1[0])
    w1r = [jnp.transpose(w1r0.reshape(2, LANES, HID), (0, 1, 2))]
    for i in range(1, NLAYERS - 1):
        w1r.append(mlp_W1[i].reshape(HID // LANES, LANES, HID))

    pw0 = jnp.zeros((2 * LANES, OUT), f32).at[:POS + 1, :].set(pred_W[0])
    pw0r = pw0.reshape(2, LANES, OUT)

    gid2 = graph_ids.reshape(GRID_N, 1, BN_BLK)
    sumb = r1(sum(pred_b[1:], pred_b[0]))

    h_c = h0c
    y = _y0(h0c, pw0r)
    for i in range(NLAYERS - 1):
        nch = h_c.shape[0]
        agg = _sc_aggregate(nch, h_c, src_b, dst_b, wb_b, zero_rows)
        u, st1 = _p1(h_c, agg, w1r[i], r1(mlp_b1[i]))
        v, st2 = _p2(u, st1, r1(mlp_g1[i]), r1(mlp_be1[i]),
                     mlp_W2[i], r1(mlp_b2[i]))
        x2, st3 = _p3(v, st2, r1(app_g[i]), r1(app_be[i]))
        h_c, y = _p4(x2, st3, r1(out_g[i]), r1(out_be[i]), pred_W[i + 1], y)

    return _readout(y, gid2, sumb)


# speedup vs baseline: 4.7109x; 1.3509x over previous
import functools

import jax
import jax.numpy as jnp
from jax import lax
from jax.experimental import pallas as pl
from jax.experimental.pallas import tpu as pltpu
from jax.experimental.pallas import tpu_sc as plsc

N = 10000
E = 160000
POS = 128
HID = 512
OUT = 128
NLAYERS = 6
NGRAPH = 256
BN_EPS = 1e-5

LANES = 128
NTILES = 16
EPT = E // NTILES
BLK = 40
RDEP = 3
IDEP = 6
UNROLL = 6
NBLK = ((EPT + BLK * UNROLL - 1) // (BLK * UNROLL)) * UNROLL
NGRP = NBLK // UNROLL
EPT_PAD = NBLK * BLK
N_PAD = 10240
ROWS_PER_TILE = N_PAD // NTILES
BN_BLK = 1000
GRID_N = N // BN_BLK


@functools.partial(jax.jit, static_argnums=(0,))
def _sc_aggregate(nch, h_c, src_b, dst_b, wb_b, zero_rows):
    mesh = plsc.VectorSubcoreMesh(core_axis_name="c", subcore_axis_name="s")

    @functools.partial(
        pl.kernel,
        mesh=mesh,
        out_type=jax.ShapeDtypeStruct((nch, N_PAD, LANES), jnp.float32),
        scratch_types=[
            pltpu.VMEM((IDEP, BLK), jnp.int32),
            pltpu.VMEM((IDEP, BLK), jnp.int32),
            pltpu.VMEM((IDEP, BLK), jnp.float32),
            pltpu.VMEM((RDEP, BLK, LANES), jnp.float32),
            pltpu.VMEM_SHARED((N_PAD, LANES), jnp.float32),
            pltpu.SemaphoreType.DMA,
            pltpu.SemaphoreType.DMA,
            pltpu.SemaphoreType.DMA,
            pltpu.SemaphoreType.DMA,
            pltpu.SemaphoreType.DMA,
        ],
    )
    def agg_kernel(h_hbm, src_hbm, dst_hbm, wb_hbm, zero_hbm, agg_hbm,
                   srcv, dstv, wbv, rowsv, slab, sem_i, sem_g,
                   sem_s0, sem_s1, sem_s2):
        sem_s = (sem_s0, sem_s1, sem_s2)
        cid = lax.axis_index("c")
        sid = lax.axis_index("s")
        base = sid * ROWS_PER_TILE

        def one_chunk(ch, carry):
            hc = h_hbm.at[ch]
            pltpu.sync_copy(zero_hbm, slab.at[pl.ds(base, ROWS_PER_TILE)])
            plsc.subcore_barrier()

            def idx_dmas(j, s):
                return (
                    pltpu.make_async_copy(src_hbm.at[sid, j],
                                          srcv.at[s], sem_i),
                    pltpu.make_async_copy(dst_hbm.at[sid, j],
                                          dstv.at[s], sem_i),
                    pltpu.make_async_copy(wb_hbm.at[sid, j],
                                          wbv.at[s], sem_i),
                )

            def gather(j, s, rs):
                return pltpu.make_async_copy(
                    hc.at[srcv.at[s]], rowsv.at[rs], sem_g)

            def scatter(s, rs):
                return pltpu.make_async_copy(
                    rowsv.at[rs], slab.at[dstv.at[s]], sem_s[rs])

            def scale(rs, s):
                def rbody(i, c2):
                    r0 = 2 * i
                    grp = pl.multiple_of((r0 // 16) * 16, 16)
                    w16 = wbv[s, pl.ds(grp, 16)]
                    dn = lax.GatherDimensionNumbers(
                        offset_dims=(), collapsed_slice_dims=(0,),
                        start_index_map=(0,))
                    for k in range(2):
                        r = r0 + k
                        wk = lax.gather(
                            w16,
                            jnp.full((16, 1), r - grp, jnp.int32), dn, (1,),
                            mode=lax.GatherScatterMode.PROMISE_IN_BOUNDS)
                        for g in range(LANES // 16):
                            sl = pl.ds(g * 16, 16)
                            rowsv[rs, r, sl] = rowsv[rs, r, sl] * wk
                    return c2

                lax.fori_loop(0, BLK // 2, rbody, 0)

            def stage(j, u, static=False):
                rs, rs1 = u % RDEP, (u + 1) % RDEP
                s, s1, s4 = u, (u + 1) % IDEP, (u + 4) % IDEP
                if not (static and j < 2):
                    scatter((u + 4) % IDEP, rs1).wait()
                if not (static and j + 1 >= NBLK):
                    for d in idx_dmas(j + 1, s1):
                        d.wait()
                    gather(j + 1, s1, rs1).start()
                gather(j, s, rs).wait()
                scale(rs, s)
                scatter(s, rs).start(add=True)
                if not (static and j + 4 >= NBLK):
                    for d in idx_dmas(j + 4, s4):
                        d.start()

            for t in range(4):
                for d in idx_dmas(t, t):
                    d.start()
            for d in idx_dmas(0, 0):
                d.wait()
            gather(0, 0, 0).start()
            for u in range(UNROLL):
                stage(u, u, static=True)

            def body(g, c2):
                for u in range(UNROLL):
                    stage(g * UNROLL + u, u)
                return c2

            lax.fori_loop(1, NGRP - 1, body, 0)
            for u in range(UNROLL):
                stage((NGRP - 1) * UNROLL + u, u, static=True)
            for j in range(NBLK - 2, NBLK):
                scatter(j % IDEP, j % RDEP).wait()

            plsc.subcore_barrier()
            pltpu.sync_copy(
                slab.at[pl.ds(base, ROWS_PER_TILE)],
                agg_hbm.at[ch].at[pl.ds(base, ROWS_PER_TILE)])
            plsc.subcore_barrier()
            return carry

        lax.fori_loop(0, nch // 2, lambda k, c2: one_chunk(cid + 2 * k, c2), 0)

    return agg_kernel(h_c, src_b, dst_b, wb_b, zero_rows)


def _p1(h_c, agg_c, w1r, b1):
    nch = h_c.shape[0]

    def body(h_ref, a_ref, w_ref, b_ref, u_ref, st_ref):
        i = pl.program_id(0)
        acc = jnp.dot(h_ref[0] + a_ref[0], w_ref[0],
                      preferred_element_type=jnp.float32)
        for c in range(1, nch):
            acc += jnp.dot(h_ref[c] + a_ref[c], w_ref[c],
                           preferred_element_type=jnp.float32)
        u = acc + b_ref[:]
        u_ref[:] = u
        blk = jnp.concatenate(
            [jnp.sum(u, axis=0, keepdims=True),
             jnp.sum(u * u, axis=0, keepdims=True)], axis=0)

        @pl.when(i == 0)
        def _():
            st_ref[:] = blk

        @pl.when(i > 0)
        def _():
            st_ref[:] = st_ref[:] + blk

    return pl.pallas_call(
        body,
        grid=(GRID_N,),
        in_specs=[
            pl.BlockSpec((nch, BN_BLK, LANES), lambda i: (0, i, 0)),
            pl.BlockSpec((nch, BN_BLK, LANES), lambda i: (0, i, 0)),
            pl.BlockSpec((nch, LANES, HID), lambda i: (0, 0, 0)),
            pl.BlockSpec((1, HID), lambda i: (0, 0)),
        ],
        out_specs=[
            pl.BlockSpec((BN_BLK, HID), lambda i: (i, 0)),
            pl.BlockSpec((2, HID), lambda i: (0, 0)),
        ],
        out_shape=[
            jax.ShapeDtypeStruct((N, HID), jnp.float32),
            jax.ShapeDtypeStruct((2, HID), jnp.float32),
        ],
    )(h_c, agg_c, w1r, b1)


def _norm(x, st_ref, g_ref, be_ref):
    m = st_ref[0:1, :] * (1.0 / N)
    var = st_ref[1:2, :] * (1.0 / N) - m * m
    xn = (x - m) * lax.rsqrt(var + BN_EPS) * g_ref[:] + be_ref[:]
    return jnp.maximum(xn, 0.0)


def _p2(u, st1, g1, be1, w2, b2):
    def body(u_ref, st_ref, g_ref, be_ref, w_ref, b_ref, v_ref, st2_ref):
        i = pl.program_id(0)
        x = _norm(u_ref[:], st_ref, g_ref, be_ref)
        v = jnp.dot(x, w_ref[:], preferred_element_type=jnp.float32) + b_ref[:]
        v_ref[:] = v
        blk = jnp.concatenate(
            [jnp.sum(v, axis=0, keepdims=True),
             jnp.sum(v * v, axis=0, keepdims=True)], axis=0)

        @pl.when(i == 0)
        def _():
            st2_ref[:] = blk

        @pl.when(i > 0)
        def _():
            st2_ref[:] = st2_ref[:] + blk

    return pl.pallas_call(
        body,
        grid=(GRID_N,),
        in_specs=[
            pl.BlockSpec((BN_BLK, HID), lambda i: (i, 0)),
            pl.BlockSpec((2, HID), lambda i: (0, 0)),
            pl.BlockSpec((1, HID), lambda i: (0, 0)),
            pl.BlockSpec((1, HID), lambda i: (0, 0)),
            pl.BlockSpec((HID, HID), lambda i: (0, 0)),
            pl.BlockSpec((1, HID), lambda i: (0, 0)),
        ],
        out_specs=[
            pl.BlockSpec((BN_BLK, HID), lambda i: (i, 0)),
            pl.BlockSpec((2, HID), lambda i: (0, 0)),
        ],
        out_shape=[
            jax.ShapeDtypeStruct((N, HID), jnp.float32),
            jax.ShapeDtypeStruct((2, HID), jnp.float32),
        ],
    )(u, st1, g1, be1, w2, b2)


def _p3(v, st2, ag, abe):
    def body(v_ref, st_ref, g_ref, be_ref, x2_ref, st3_ref):
        i = pl.program_id(0)
        x2 = _norm(v_ref[:], st_ref, g_ref, be_ref)
        x2_ref[:] = x2
        blk = jnp.concatenate(
            [jnp.sum(x2, axis=0, keepdims=True),
             jnp.sum(x2 * x2, axis=0, keepdims=True)], axis=0)

        @pl.when(i == 0)
        def _():
            st3_ref[:] = blk

        @pl.when(i > 0)
        def _():
            st3_ref[:] = st3_ref[:] + blk

    return pl.pallas_call(
        body,
        grid=(GRID_N,),
        in_specs=[
            pl.BlockSpec((BN_BLK, HID), lambda i: (i, 0)),
            pl.BlockSpec((2, HID), lambda i: (0, 0)),
            pl.BlockSpec((1, HID), lambda i: (0, 0)),
            pl.BlockSpec((1, HID), lambda i: (0, 0)),
        ],
        out_specs=[
            pl.BlockSpec((BN_BLK, HID), lambda i: (i, 0)),
            pl.BlockSpec((2, HID), lambda i: (0, 0)),
        ],
        out_shape=[
            jax.ShapeDtypeStruct((N, HID), jnp.float32),
            jax.ShapeDtypeStruct((2, HID), jnp.float32),
        ],
    )(v, st2, ag, abe)


def _p4(x2, st3, og, obe, pw, y_in):
    nch = HID // LANES

    def body(x_ref, st_ref, g_ref, be_ref, pw_ref, y_ref, h4_ref, yo_ref):
        hn = _norm(x_ref[:], st_ref, g_ref, be_ref)
        for c in range(nch):
            h4_ref[c] = hn[:, c * LANES:(c + 1) * LANES]
        yo_ref[:] = y_ref[:] + jnp.dot(hn, pw_ref[:],
                                       preferred_element_type=jnp.float32)

    return pl.pallas_call(
        body,
        grid=(GRID_N,),
        in_specs=[
            pl.BlockSpec((BN_BLK, HID), lambda i: (i, 0)),
            pl.BlockSpec((2, HID), lambda i: (0, 0)),
            pl.BlockSpec((1, HID), lambda i: (0, 0)),
            pl.BlockSpec((1, HID), lambda i: (0, 0)),
            pl.BlockSpec((HID, OUT), lambda i: (0, 0)),
            pl.BlockSpec((BN_BLK, OUT), lambda i: (i, 0)),
        ],
        out_specs=[
            pl.BlockSpec((nch, BN_BLK, LANES), lambda i: (0, i, 0)),
            pl.BlockSpec((BN_BLK, OUT), lambda i: (i, 0)),
        ],
        out_shape=[
            jax.ShapeDtypeStruct((nch, N, LANES), jnp.float32),
            jax.ShapeDtypeStruct((N, OUT), jnp.float32),
        ],
        input_output_aliases={5: 1},
    )(x2, st3, og, obe, pw, y_in)


def _y0(h0c, pw0r):
    nch = h0c.shape[0]

    def body(h_ref, w_ref, y_ref):
        acc = jnp.dot(h_ref[0], w_ref[0], preferred_element_type=jnp.float32)
        for c in range(1, nch):
            acc += jnp.dot(h_ref[c], w_ref[c],
                           preferred_element_type=jnp.float32)
        y_ref[:] = acc

    return pl.pallas_call(
        body,
        grid=(GRID_N,),
        in_specs=[
            pl.BlockSpec((nch, BN_BLK, LANES), lambda i: (0, i, 0)),
            pl.BlockSpec((nch, LANES, OUT), lambda i: (0, 0, 0)),
        ],
        out_specs=pl.BlockSpec((BN_BLK, OUT), lambda i: (i, 0)),
        out_shape=jax.ShapeDtypeStruct((N, OUT), jnp.float32),
    )(h0c, pw0r)


def _readout(y, gid2, sumb):
    def body(y_ref, g_ref, sb_ref, s_ref):
        i = pl.program_id(0)
        seg = lax.broadcasted_iota(jnp.int32, (NGRAPH, BN_BLK), 0)
        p = (g_ref[0] == seg).astype(jnp.float32)
        blk = jnp.dot(p, y_ref[:], preferred_element_type=jnp.float32)

        @pl.when(i == 0)
        def _():
            s_ref[:] = blk + sb_ref[:]

        @pl.when(i > 0)
        def _():
            s_ref[:] = s_ref[:] + blk

    return pl.pallas_call(
        body,
        grid=(GRID_N,),
        in_specs=[
            pl.BlockSpec((BN_BLK, OUT), lambda i: (i, 0)),
            pl.BlockSpec((1, 1, BN_BLK), lambda i: (i, 0, 0)),
            pl.BlockSpec((1, OUT), lambda i: (0, 0)),
        ],
        out_specs=pl.BlockSpec((NGRAPH, OUT), lambda i: (0, 0)),
        out_shape=jax.ShapeDtypeStruct((NGRAPH, OUT), jnp.float32),
    )(y, gid2, sumb)


def kernel(pos_undirected, seed, edge_index, edge_weight, graph_ids,
           mlp_W1, mlp_b1, mlp_g1, mlp_be1, mlp_W2, mlp_b2,
           app_g, app_be, out_g, out_be, pred_W, pred_b):
    f32 = jnp.float32

    h0 = jnp.concatenate([pos_undirected, seed.astype(f32)[:, None]], axis=1)
    h0p = jnp.pad(h0, ((0, 0), (0, 2 * LANES - (POS + 1))))
    h0c = jnp.transpose(h0p.reshape(N, 2, LANES), (1, 0, 2))

    pad = EPT_PAD * NTILES - E
    ar = jnp.arange(pad, dtype=jnp.int32)
    src_b = jnp.concatenate([edge_index[0], ar]).reshape(NTILES, NBLK, BLK)
    dst_b = jnp.concatenate([edge_index[1], ar]).reshape(NTILES, NBLK, BLK)
    w_p = jnp.concatenate([edge_weight, jnp.zeros((pad,), f32)])
    wb_b = w_p.reshape(NTILES, NBLK, BLK)
    zero_rows = jnp.zeros((ROWS_PER_TILE, LANES), f32)

    def r1(x):
        return x.reshape(1, -1)

    w1r0 = jnp.zeros((2 * LANES, HID), f32).at[:POS + 1, :].set(mlp_W1[0])
    w1r = [jnp.transpose(w1r0.reshape(2, LANES, HID), (0, 1, 2))]
    for i in range(1, NLAYERS - 1):
        w1r.append(mlp_W1[i].reshape(HID // LANES, LANES, HID))

    pw0 = jnp.zeros((2 * LANES, OUT), f32).at[:POS + 1, :].set(pred_W[0])
    pw0r = pw0.reshape(2, LANES, OUT)

    gid2 = graph_ids.reshape(GRID_N, 1, BN_BLK)
    sumb = r1(sum(pred_b[1:], pred_b[0]))

    h_c = h0c
    y = _y0(h0c, pw0r)
    for i in range(NLAYERS - 1):
        nch = h_c.shape[0]
        agg = _sc_aggregate(nch, h_c, src_b, dst_b, wb_b, zero_rows)
        u, st1 = _p1(h_c, agg, w1r[i], r1(mlp_b1[i]))
        v, st2 = _p2(u, st1, r1(mlp_g1[i]), r1(mlp_be1[i]),
                     mlp_W2[i], r1(mlp_b2[i]))
        x2, st3 = _p3(v, st2, r1(app_g[i]), r1(app_be[i]))
        h_c, y = _p4(x2, st3, r1(out_g[i]), r1(out_be[i]), pred_W[i + 1], y)

    return _readout(y, gid2, sumb)


# speedup vs baseline: 4.8281x; 1.0249x over previous
import functools

import jax
import jax.numpy as jnp
from jax import lax
from jax.experimental import pallas as pl
from jax.experimental.pallas import tpu as pltpu
from jax.experimental.pallas import tpu_sc as plsc

N = 10000
E = 160000
POS = 128
HID = 512
OUT = 128
NLAYERS = 6
NGRAPH = 256
BN_EPS = 1e-5

LANES = 128
NTILES = 16
EPT = E // NTILES
BLK = 40
RDEP = 3
IDEP = 6
UNROLL = 6
NBLK = ((EPT + BLK * UNROLL - 1) // (BLK * UNROLL)) * UNROLL
NGRP = NBLK // UNROLL
EPT_PAD = NBLK * BLK
N_PAD = 10240
ROWS_PER_TILE = N_PAD // NTILES
BN_BLK = 1000
GRID_N = N // BN_BLK


@functools.partial(jax.jit, static_argnums=(0,))
def _sc_aggregate(nch, h_c, src_b, dst_b, wb_b, zero_rows):
    mesh = plsc.VectorSubcoreMesh(core_axis_name="c", subcore_axis_name="s")

    @functools.partial(
        pl.kernel,
        mesh=mesh,
        out_type=jax.ShapeDtypeStruct((nch, N_PAD, LANES), jnp.float32),
        scratch_types=[
            pltpu.VMEM((IDEP, BLK), jnp.int32),
            pltpu.VMEM((IDEP, BLK), jnp.int32),
            pltpu.VMEM((IDEP, BLK), jnp.float32),
            pltpu.VMEM((RDEP, BLK, LANES), jnp.float32),
            pltpu.VMEM_SHARED((N_PAD, LANES), jnp.float32),
            pltpu.SemaphoreType.DMA,
            pltpu.SemaphoreType.DMA,
            pltpu.SemaphoreType.DMA,
            pltpu.SemaphoreType.DMA,
            pltpu.SemaphoreType.DMA,
        ],
    )
    def agg_kernel(h_hbm, src_hbm, dst_hbm, wb_hbm, zero_hbm, agg_hbm,
                   srcv, dstv, wbv, rowsv, slab, sem_i, sem_g,
                   sem_s0, sem_s1, sem_s2):
        sem_s = (sem_s0, sem_s1, sem_s2)
        cid = lax.axis_index("c")
        sid = lax.axis_index("s")
        base = sid * ROWS_PER_TILE

        def one_chunk(ch, carry):
            hc = h_hbm.at[ch]
            pltpu.sync_copy(zero_hbm, slab.at[pl.ds(base, ROWS_PER_TILE)])
            plsc.subcore_barrier()

            def idx_dmas(j, s):
                return (
                    pltpu.make_async_copy(src_hbm.at[sid, j],
                                          srcv.at[s], sem_i),
                    pltpu.make_async_copy(dst_hbm.at[sid, j],
                                          dstv.at[s], sem_i),
                    pltpu.make_async_copy(wb_hbm.at[sid, j],
                                          wbv.at[s], sem_i),
                )

            def gather(j, s, rs):
                return pltpu.make_async_copy(
                    hc.at[srcv.at[s]], rowsv.at[rs], sem_g)

            def scatter(s, rs):
                return pltpu.make_async_copy(
                    rowsv.at[rs], slab.at[dstv.at[s]], sem_s[rs])

            def scale(rs, s):
                def rbody(i, c2):
                    r0 = 4 * i
                    grp = pl.multiple_of((r0 // 16) * 16, 16)
                    w16 = wbv[s, pl.ds(grp, 16)]
                    dn = lax.GatherDimensionNumbers(
                        offset_dims=(), collapsed_slice_dims=(0,),
                        start_index_map=(0,))
                    for k in range(4):
                        r = r0 + k
                        wk = lax.gather(
                            w16,
                            jnp.full((16, 1), r - grp, jnp.int32), dn, (1,),
                            mode=lax.GatherScatterMode.PROMISE_IN_BOUNDS)
                        for g in range(LANES // 16):
                            sl = pl.ds(g * 16, 16)
                            rowsv[rs, r, sl] = rowsv[rs, r, sl] * wk
                    return c2

                lax.fori_loop(0, BLK // 4, rbody, 0)

            def stage(j, u, static=False):
                rs, rs1 = u % RDEP, (u + 1) % RDEP
                s, s1, s4 = u, (u + 1) % IDEP, (u + 4) % IDEP
                if not (static and j < 2):
                    scatter((u + 4) % IDEP, rs1).wait()
                if not (static and j + 1 >= NBLK):
                    for d in idx_dmas(j + 1, s1):
                        d.wait()
                    gather(j + 1, s1, rs1).start()
                gather(j, s, rs).wait()
                scale(rs, s)
                scatter(s, rs).start(add=True)
                if not (static and j + 4 >= NBLK):
                    for d in idx_dmas(j + 4, s4):
                        d.start()

            for t in range(4):
                for d in idx_dmas(t, t):
                    d.start()
            for d in idx_dmas(0, 0):
                d.wait()
            gather(0, 0, 0).start()
            for u in range(UNROLL):
                stage(u, u, static=True)

            def body(g, c2):
                for u in range(UNROLL):
                    stage(g * UNROLL + u, u)
                return c2

            lax.fori_loop(1, NGRP - 1, body, 0)
            for u in range(UNROLL):
                stage((NGRP - 1) * UNROLL + u, u, static=True)
            for j in range(NBLK - 2, NBLK):
                scatter(j % IDEP, j % RDEP).wait()

            plsc.subcore_barrier()
            pltpu.sync_copy(
                slab.at[pl.ds(base, ROWS_PER_TILE)],
                agg_hbm.at[ch].at[pl.ds(base, ROWS_PER_TILE)])
            plsc.subcore_barrier()
            return carry

        lax.fori_loop(0, nch // 2, lambda k, c2: one_chunk(cid + 2 * k, c2), 0)

    return agg_kernel(h_c, src_b, dst_b, wb_b, zero_rows)


def _p1(h_c, agg_c, w1r, b1):
    nch = h_c.shape[0]

    def body(h_ref, a_ref, w_ref, b_ref, u_ref, st_ref):
        i = pl.program_id(0)
        acc = jnp.dot(h_ref[0] + a_ref[0], w_ref[0],
                      preferred_element_type=jnp.float32)
        for c in range(1, nch):
            acc += jnp.dot(h_ref[c] + a_ref[c], w_ref[c],
                           preferred_element_type=jnp.float32)
        u = acc + b_ref[:]
        u_ref[:] = u
        blk = jnp.concatenate(
            [jnp.sum(u, axis=0, keepdims=True),
             jnp.sum(u * u, axis=0, keepdims=True)], axis=0)

        @pl.when(i == 0)
        def _():
            st_ref[:] = blk

        @pl.when(i > 0)
        def _():
            st_ref[:] = st_ref[:] + blk

    return pl.pallas_call(
        body,
        grid=(GRID_N,),
        in_specs=[
            pl.BlockSpec((nch, BN_BLK, LANES), lambda i: (0, i, 0)),
            pl.BlockSpec((nch, BN_BLK, LANES), lambda i: (0, i, 0)),
            pl.BlockSpec((nch, LANES, HID), lambda i: (0, 0, 0)),
            pl.BlockSpec((1, HID), lambda i: (0, 0)),
        ],
        out_specs=[
            pl.BlockSpec((BN_BLK, HID), lambda i: (i, 0)),
            pl.BlockSpec((2, HID), lambda i: (0, 0)),
        ],
        out_shape=[
            jax.ShapeDtypeStruct((N, HID), jnp.float32),
            jax.ShapeDtypeStruct((2, HID), jnp.float32),
        ],
    )(h_c, agg_c, w1r, b1)


def _norm(x, st_ref, g_ref, be_ref):
    m = st_ref[0:1, :] * (1.0 / N)
    var = st_ref[1:2, :] * (1.0 / N) - m * m
    xn = (x - m) * lax.rsqrt(var + BN_EPS) * g_ref[:] + be_ref[:]
    return jnp.maximum(xn, 0.0)


def _p2(u, st1, g1, be1, w2, b2):
    def body(u_ref, st_ref, g_ref, be_ref, w_ref, b_ref, v_ref, st2_ref):
        i = pl.program_id(0)
        x = _norm(u_ref[:], st_ref, g_ref, be_ref)
        v = jnp.dot(x, w_ref[:], preferred_element_type=jnp.float32) + b_ref[:]
        v_ref[:] = v
        blk = jnp.concatenate(
            [jnp.sum(v, axis=0, keepdims=True),
             jnp.sum(v * v, axis=0, keepdims=True)], axis=0)

        @pl.when(i == 0)
        def _():
            st2_ref[:] = blk

        @pl.when(i > 0)
        def _():
            st2_ref[:] = st2_ref[:] + blk

    return pl.pallas_call(
        body,
        grid=(GRID_N,),
        in_specs=[
            pl.BlockSpec((BN_BLK, HID), lambda i: (i, 0)),
            pl.BlockSpec((2, HID), lambda i: (0, 0)),
            pl.BlockSpec((1, HID), lambda i: (0, 0)),
            pl.BlockSpec((1, HID), lambda i: (0, 0)),
            pl.BlockSpec((HID, HID), lambda i: (0, 0)),
            pl.BlockSpec((1, HID), lambda i: (0, 0)),
        ],
        out_specs=[
            pl.BlockSpec((BN_BLK, HID), lambda i: (i, 0)),
            pl.BlockSpec((2, HID), lambda i: (0, 0)),
        ],
        out_shape=[
            jax.ShapeDtypeStruct((N, HID), jnp.float32),
            jax.ShapeDtypeStruct((2, HID), jnp.float32),
        ],
    )(u, st1, g1, be1, w2, b2)


def _p3(v, st2, ag, abe):
    def body(v_ref, st_ref, g_ref, be_ref, x2_ref, st3_ref):
        i = pl.program_id(0)
        x2 = _norm(v_ref[:], st_ref, g_ref, be_ref)
        x2_ref[:] = x2
        blk = jnp.concatenate(
            [jnp.sum(x2, axis=0, keepdims=True),
             jnp.sum(x2 * x2, axis=0, keepdims=True)], axis=0)

        @pl.when(i == 0)
        def _():
            st3_ref[:] = blk

        @pl.when(i > 0)
        def _():
            st3_ref[:] = st3_ref[:] + blk

    return pl.pallas_call(
        body,
        grid=(GRID_N,),
        in_specs=[
            pl.BlockSpec((BN_BLK, HID), lambda i: (i, 0)),
            pl.BlockSpec((2, HID), lambda i: (0, 0)),
            pl.BlockSpec((1, HID), lambda i: (0, 0)),
            pl.BlockSpec((1, HID), lambda i: (0, 0)),
        ],
        out_specs=[
            pl.BlockSpec((BN_BLK, HID), lambda i: (i, 0)),
            pl.BlockSpec((2, HID), lambda i: (0, 0)),
        ],
        out_shape=[
            jax.ShapeDtypeStruct((N, HID), jnp.float32),
            jax.ShapeDtypeStruct((2, HID), jnp.float32),
        ],
    )(v, st2, ag, abe)


def _p4(x2, st3, og, obe, pw, y_in):
    nch = HID // LANES

    def body(x_ref, st_ref, g_ref, be_ref, pw_ref, y_ref, h4_ref, yo_ref):
        hn = _norm(x_ref[:], st_ref, g_ref, be_ref)
        for c in range(nch):
            h4_ref[c] = hn[:, c * LANES:(c + 1) * LANES]
        yo_ref[:] = y_ref[:] + jnp.dot(hn, pw_ref[:],
                                       preferred_element_type=jnp.float32)

    return pl.pallas_call(
        body,
        grid=(GRID_N,),
        in_specs=[
            pl.BlockSpec((BN_BLK, HID), lambda i: (i, 0)),
            pl.BlockSpec((2, HID), lambda i: (0, 0)),
            pl.BlockSpec((1, HID), lambda i: (0, 0)),
            pl.BlockSpec((1, HID), lambda i: (0, 0)),
            pl.BlockSpec((HID, OUT), lambda i: (0, 0)),
            pl.BlockSpec((BN_BLK, OUT), lambda i: (i, 0)),
        ],
        out_specs=[
            pl.BlockSpec((nch, BN_BLK, LANES), lambda i: (0, i, 0)),
            pl.BlockSpec((BN_BLK, OUT), lambda i: (i, 0)),
        ],
        out_shape=[
            jax.ShapeDtypeStruct((nch, N, LANES), jnp.float32),
            jax.ShapeDtypeStruct((N, OUT), jnp.float32),
        ],
        input_output_aliases={5: 1},
    )(x2, st3, og, obe, pw, y_in)


def _y0(h0c, pw0r):
    nch = h0c.shape[0]

    def body(h_ref, w_ref, y_ref):
        acc = jnp.dot(h_ref[0], w_ref[0], preferred_element_type=jnp.float32)
        for c in range(1, nch):
            acc += jnp.dot(h_ref[c], w_ref[c],
                           preferred_element_type=jnp.float32)
        y_ref[:] = acc

    return pl.pallas_call(
        body,
        grid=(GRID_N,),
        in_specs=[
            pl.BlockSpec((nch, BN_BLK, LANES), lambda i: (0, i, 0)),
            pl.BlockSpec((nch, LANES, OUT), lambda i: (0, 0, 0)),
        ],
        out_specs=pl.BlockSpec((BN_BLK, OUT), lambda i: (i, 0)),
        out_shape=jax.ShapeDtypeStruct((N, OUT), jnp.float32),
    )(h0c, pw0r)


def _readout(y, gid2, sumb):
    def body(y_ref, g_ref, sb_ref, s_ref):
        i = pl.program_id(0)
        seg = lax.broadcasted_iota(jnp.int32, (NGRAPH, BN_BLK), 0)
        p = (g_ref[0] == seg).astype(jnp.float32)
        blk = jnp.dot(p, y_ref[:], preferred_element_type=jnp.float32)

        @pl.when(i == 0)
        def _():
            s_ref[:] = blk + sb_ref[:]

        @pl.when(i > 0)
        def _():
            s_ref[:] = s_ref[:] + blk

    return pl.pallas_call(
        body,
        grid=(GRID_N,),
        in_specs=[
            pl.BlockSpec((BN_BLK, OUT), lambda i: (i, 0)),
            pl.BlockSpec((1, 1, BN_BLK), lambda i: (i, 0, 0)),
            pl.BlockSpec((1, OUT), lambda i: (0, 0)),
        ],
        out_specs=pl.BlockSpec((NGRAPH, OUT), lambda i: (0, 0)),
        out_shape=jax.ShapeDtypeStruct((NGRAPH, OUT), jnp.float32),
    )(y, gid2, sumb)


def kernel(pos_undirected, seed, edge_index, edge_weight, graph_ids,
           mlp_W1, mlp_b1, mlp_g1, mlp_be1, mlp_W2, mlp_b2,
           app_g, app_be, out_g, out_be, pred_W, pred_b):
    f32 = jnp.float32

    h0 = jnp.concatenate([pos_undirected, seed.astype(f32)[:, None]], axis=1)
    h0p = jnp.pad(h0, ((0, 0), (0, 2 * LANES - (POS + 1))))
    h0c = jnp.transpose(h0p.reshape(N, 2, LANES), (1, 0, 2))

    pad = EPT_PAD * NTILES - E
    ar = jnp.arange(pad, dtype=jnp.int32)
    src_b = jnp.concatenate([edge_index[0], ar]).reshape(NTILES, NBLK, BLK)
    dst_b = jnp.concatenate([edge_index[1], ar]).reshape(NTILES, NBLK, BLK)
    w_p = jnp.concatenate([edge_weight, jnp.zeros((pad,), f32)])
    wb_b = w_p.reshape(NTILES, NBLK, BLK)
    zero_rows = jnp.zeros((ROWS_PER_TILE, LANES), f32)

    def r1(x):
        return x.reshape(1, -1)

    w1r0 = jnp.zeros((2 * LANES, HID), f32).at[:POS + 1, :].set(mlp_W1[0])
    w1r = [jnp.transpose(w1r0.reshape(2, LANES, HID), (0, 1, 2))]
    for i in range(1, NLAYERS - 1):
        w1r.append(mlp_W1[i].reshape(HID // LANES, LANES, HID))

    pw0 = jnp.zeros((2 * LANES, OUT), f32).at[:POS + 1, :].set(pred_W[0])
    pw0r = pw0.reshape(2, LANES, OUT)

    gid2 = graph_ids.reshape(GRID_N, 1, BN_BLK)
    sumb = r1(sum(pred_b[1:], pred_b[0]))

    h_c = h0c
    y = _y0(h0c, pw0r)
    for i in range(NLAYERS - 1):
        nch = h_c.shape[0]
        agg = _sc_aggregate(nch, h_c, src_b, dst_b, wb_b, zero_rows)
        u, st1 = _p1(h_c, agg, w1r[i], r1(mlp_b1[i]))
        v, st2 = _p2(u, st1, r1(mlp_g1[i]), r1(mlp_be1[i]),
                     mlp_W2[i], r1(mlp_b2[i]))
        x2, st3 = _p3(v, st2, r1(app_g[i]), r1(app_be[i]))
        h_c, y = _p4(x2, st3, r1(out_g[i]), r1(out_be[i]), pred_W[i + 1], y)

    return _readout(y, gid2, sumb)
